# baseline (speedup 1.0000x reference)
import functools

import jax
import jax.numpy as jnp
from jax import lax
from jax.experimental import pallas as pl
from jax.experimental.pallas import tpu as pltpu
from jax.experimental.pallas import tpu_sc as plsc

NC = 2
NS = 16
NW = NC * NS
CH = 80


def _sc_mesh():
    return plsc.VectorSubcoreMesh(
        core_axis_name="c", subcore_axis_name="s", num_cores=NC, num_subcores=NS
    )


def _deg_call(dst3, zeros1, n_sp, nchunk):
    rpt = n_sp // NS

    def body(dst_hbm, z_hbm, out_hbm, dst_v, ones_v, deg_sh):
        c = lax.axis_index("c")
        s = lax.axis_index("s")
        w = s * NC + c
        pltpu.sync_copy(z_hbm.at[pl.ds(s * rpt, rpt)], deg_sh.at[pl.ds(s * rpt, rpt)])
        for i in range(CH // 16):
            ones_v[pl.ds(i * 16, 16)] = jnp.full((16,), 1.0, jnp.float32)
        pltpu.sync_copy(dst_hbm.at[w], dst_v)
        plsc.subcore_barrier()

        def chunk(j, carry):
            pltpu.sync_copy(ones_v, deg_sh.at[dst_v.at[j]], add=True)
            return carry

        lax.fori_loop(0, nchunk, chunk, 0)
        plsc.subcore_barrier()
        pltpu.sync_copy(deg_sh.at[pl.ds(s * rpt, rpt)],
                        out_hbm.at[c].at[pl.ds(s * rpt, rpt)])

    f = pl.kernel(
        body,
        out_type=jax.ShapeDtypeStruct((NC, n_sp), jnp.float32),
        mesh=_sc_mesh(),
        scratch_types=[
            pltpu.VMEM((nchunk, CH), jnp.int32),
            pltpu.VMEM((CH,), jnp.float32),
            pltpu.VMEM_SHARED((n_sp,), jnp.float32),
        ],
    )
    return f(dst3, zeros1)


def _spmm_call(pk0, pk1, hs, zeros2, n_sp, d, n0, n1):
    rpt = n_sp // NS

    def body(pk0_hbm, pk1_hbm, hs_hbm, z_hbm, out_hbm, pk_v, sidx_v, didx_v,
             rows_v, agg_sh, gsem, ssem, psem0, psem1):
        c = lax.axis_index("c")
        s = lax.axis_index("s")
        sl = pl.ds(s * rpt, rpt)
        psem = (psem0, psem1)

        @pl.when(c == 0)
        def _():
            pltpu.sync_copy(z_hbm.at[sl], agg_sh.at[sl])

        @pl.when(c != 0)
        def _():
            pltpu.sync_copy(hs_hbm.at[sl], agg_sh.at[sl])

        plsc.subcore_barrier()

        def run(pk_hbm, nck):
            def pkload(j, m):
                pltpu.async_copy(pk_hbm.at[s].at[j], pk_v.at[m], psem[m])

            def pk_wait(j, m):
                pltpu.make_async_copy(pk_hbm.at[s].at[j], pk_v.at[m], psem[m]).wait()

            def unpack_idx(b, m):
                for i in range(CH // 16):
                    pkw = pk_v[m, pl.ds(i * 16, 16)]
                    sidx_v[b, pl.ds(i * 16, 16)] = lax.bitwise_and(pkw, 0xFFFF)
                    didx_v[b, pl.ds(i * 16, 16)] = lax.shift_right_logical(pkw, 16)

            def gather(b):
                pltpu.async_copy(hs_hbm.at[sidx_v.at[b]], rows_v.at[b], gsem)

            def gather_wait(b):
                pltpu.make_async_copy(hs_hbm.at[sidx_v.at[b]], rows_v.at[b], gsem).wait()

            def scatter(b):
                pltpu.async_copy(rows_v.at[b], agg_sh.at[didx_v.at[b]], ssem, add=True)

            def scatter_wait(b):
                pltpu.make_async_copy(rows_v.at[b], agg_sh.at[didx_v.at[b]], ssem).wait()

            pltpu.sync_copy(pk_hbm.at[s].at[0], pk_v.at[0])
            pltpu.sync_copy(pk_hbm.at[s].at[1], pk_v.at[1])
            unpack_idx(0, 0)
            unpack_idx(1, 1)
            gather(0)
            gather(1)
            pkload(2, 0)
            pkload(3, 1)

            def quad(q, carry):
                for u in range(4):
                    j = q * 4 + u
                    b = u % 4
                    nb = (u + 2) % 4
                    m = (u + 2) % 2

                    @pl.when(j >= 2)
                    def _():
                        scatter_wait(nb)

                    @pl.when(j + 2 < nck)
                    def _():
                        pk_wait(j + 2, m)
                        unpack_idx(nb, m)
                        gather(nb)

                    @pl.when(j + 4 < nck)
                    def _():
                        pkload(j + 4, m)

                    gather_wait(b)
                    scatter(b)
                return carry

            lax.fori_loop(0, nck // 4, quad, 0)
            scatter_wait((nck - 2) % 4)
            scatter_wait((nck - 1) % 4)

        @pl.when(c == 0)
        def _():
            run(pk0_hbm, n0)

        @pl.when(c != 0)
        def _():
            run(pk1_hbm, n1)

        plsc.subcore_barrier()
        pltpu.sync_copy(agg_sh.at[sl], out_hbm.at[c].at[sl])

    f = pl.kernel(
        body,
        out_type=jax.ShapeDtypeStruct((NC, n_sp, d), jnp.float32),
        mesh=_sc_mesh(),
        scratch_types=[
            pltpu.VMEM((2, CH), jnp.int32),
            pltpu.VMEM((4, CH), jnp.int32),
            pltpu.VMEM((4, CH), jnp.int32),
            pltpu.VMEM((4, CH, d), jnp.float32),
            pltpu.VMEM_SHARED((n_sp, d), jnp.float32),
            pltpu.SemaphoreType.DMA,
            pltpu.SemaphoreType.DMA,
            pltpu.SemaphoreType.DMA,
            pltpu.SemaphoreType.DMA,
        ],
    )
    return f(pk0, pk1, hs, zeros2)


def _tc_prologue(dega, degb, maskc, xp, W, n_sp, d, bn):
    def body(dega_ref, degb_ref, mask_ref, x_ref, w_ref, dinv_ref, hs_ref):
        deg = dega_ref[...] + degb_ref[...] + 1.0
        dinv = lax.rsqrt(deg) * mask_ref[...]
        dinv_ref[...] = dinv
        hs_ref[...] = jnp.dot(x_ref[...], w_ref[...],
                              preferred_element_type=jnp.float32) * dinv

    return pl.pallas_call(
        body,
        grid=(n_sp // bn,),
        in_specs=[
            pl.BlockSpec((bn, 1), lambda i: (i, 0)),
            pl.BlockSpec((bn, 1), lambda i: (i, 0)),
            pl.BlockSpec((bn, 1), lambda i: (i, 0)),
            pl.BlockSpec((bn, d), lambda i: (i, 0)),
            pl.BlockSpec((d, d), lambda i: (0, 0)),
        ],
        out_specs=[
            pl.BlockSpec((bn, 1), lambda i: (i, 0)),
            pl.BlockSpec((bn, d), lambda i: (i, 0)),
        ],
        out_shape=[
            jax.ShapeDtypeStruct((n_sp, 1), jnp.float32),
            jax.ShapeDtypeStruct((n_sp, d), jnp.float32),
        ],
    )(dega, degb, maskc, xp, W)


def _tc_mid(agg, dinv, b2d, W, n_sp, d, bn):
    def body(agga_ref, aggb_ref, dinv_ref, b_ref, w_ref, out_ref):
        g = (agga_ref[...] + aggb_ref[...]) * dinv_ref[...] + b_ref[...]
        h = jnp.maximum(g, 0.0)
        out_ref[...] = jnp.dot(h, w_ref[...],
                               preferred_element_type=jnp.float32) * dinv_ref[...]

    return pl.pallas_call(
        body,
        grid=(n_sp // bn,),
        in_specs=[
            pl.BlockSpec((bn, d), lambda i: (i, 0)),
            pl.BlockSpec((bn, d), lambda i: (i, 0)),
            pl.BlockSpec((bn, 1), lambda i: (i, 0)),
            pl.BlockSpec((1, d), lambda i: (0, 0)),
            pl.BlockSpec((d, d), lambda i: (0, 0)),
        ],
        out_specs=pl.BlockSpec((bn, d), lambda i: (i, 0)),
        out_shape=jax.ShapeDtypeStruct((n_sp, d), jnp.float32),
    )(agg[0], agg[1], dinv, b2d, W)


def _tc_epilogue(agg, dinv, maskc, b2d, Wl, bl2d, n_sp, d, n_real, bn):
    grid_n = n_sp // bn

    def body(agga_ref, aggb_ref, dinv_ref, mask_ref, b_ref, wl_ref, bl_ref, out_ref):
        i = pl.program_id(0)
        g = (agga_ref[...] + aggb_ref[...]) * dinv_ref[...] + b_ref[...]
        h = jnp.maximum(g, 0.0) * mask_ref[...]
        psum = jnp.sum(h, axis=0, keepdims=True)

        @pl.when(i == 0)
        def _():
            out_ref[...] = psum

        @pl.when(i > 0)
        def _():
            out_ref[...] += psum

        @pl.when(i == grid_n - 1)
        def _():
            pooled = out_ref[...] * (1.0 / n_real)
            out_ref[...] = lax.dot_general(
                pooled, wl_ref[...], (((1,), (1,)), ((), ())),
                preferred_element_type=jnp.float32) + bl_ref[...]

    return pl.pallas_call(
        body,
        grid=(grid_n,),
        in_specs=[
            pl.BlockSpec((bn, d), lambda i: (i, 0)),
            pl.BlockSpec((bn, d), lambda i: (i, 0)),
            pl.BlockSpec((bn, 1), lambda i: (i, 0)),
            pl.BlockSpec((bn, 1), lambda i: (i, 0)),
            pl.BlockSpec((1, d), lambda i: (0, 0)),
            pl.BlockSpec((d, d), lambda i: (0, 0)),
            pl.BlockSpec((1, d), lambda i: (0, 0)),
        ],
        out_specs=pl.BlockSpec((1, d), lambda i: (0, 0)),
        out_shape=jax.ShapeDtypeStruct((1, d), jnp.float32),
    )(agg[0], agg[1], dinv, maskc, b2d, Wl, bl2d)


def kernel(x, edge_index, W1, b1, W2, b2, W3, b3, Wl, bl):
    N, d = x.shape
    E = edge_index.shape[1]
    n_sp = ((N + 8 + NS * 128 - 1) // (NS * 128)) * (NS * 128)
    tchunk = -(-E // CH)
    n0 = max(4, int(round(tchunk * 0.835 / NS / 4)) * 4)
    n1 = max(4, -(-max(tchunk - n0 * NS, 1) // (NS * 4)) * 4)
    e_pad = (n0 + n1) * NS * CH
    bn = n_sp // NS
    while bn > 1600 or n_sp % bn or bn % 8:
        bn //= 2

    ei = edge_index.astype(jnp.int32)
    padv = jnp.full((e_pad - E,), N, jnp.int32)
    src_f = jnp.concatenate([ei[0], padv])
    dst_f = jnp.concatenate([ei[1], padv])
    packed2 = (src_f | (dst_f << 16)).reshape(-1, CH)
    pk0 = packed2[:NS * n0].reshape(NS, n0, CH)
    pk1 = packed2[NS * n0:].reshape(NS, n1, CH)
    nchunk = -(-tchunk // NW)
    nchunk = -(-nchunk // 2) * 2
    dst3 = jnp.concatenate(
        [dst_f[:E], jnp.full((NW * nchunk * CH - E,), N, jnp.int32)]
    ).reshape(NW, nchunk, CH)
    xp = jnp.zeros((n_sp, d), jnp.float32).at[:N, :].set(x)
    maskc = (jnp.arange(n_sp, dtype=jnp.int32) < N).astype(jnp.float32)[:, None]
    zeros2 = jnp.zeros((n_sp, d), jnp.float32)
    zeros1 = jnp.zeros((n_sp,), jnp.float32)

    deg2 = _deg_call(dst3, zeros1, n_sp, nchunk)
    dinv, hs1 = _tc_prologue(deg2[0][:, None], deg2[1][:, None], maskc, xp, W1,
                             n_sp, d, bn)
    agg1 = _spmm_call(pk0, pk1, hs1, zeros2, n_sp, d, n0, n1)
    hs2 = _tc_mid(agg1, dinv, b1[None, :], W2, n_sp, d, bn)
    agg2 = _spmm_call(pk0, pk1, hs2, zeros2, n_sp, d, n0, n1)
    hs3 = _tc_mid(agg2, dinv, b2[None, :], W3, n_sp, d, bn)
    agg3 = _spmm_call(pk0, pk1, hs3, zeros2, n_sp, d, n0, n1)
    return _tc_epilogue(agg3, dinv, maskc, b3[None, :], Wl,
                        bl[None, :], n_sp, d, N, bn)


# speedup vs baseline: 1.0555x; 1.0555x over previous
import functools

import jax
import jax.numpy as jnp
from jax import lax
from jax.experimental import pallas as pl
from jax.experimental.pallas import tpu as pltpu
from jax.experimental.pallas import tpu_sc as plsc

NC = 2
NS = 16
NW = NC * NS
CH = 112


def _sc_mesh():
    return plsc.VectorSubcoreMesh(
        core_axis_name="c", subcore_axis_name="s", num_cores=NC, num_subcores=NS
    )


def _deg_call(dst3, zeros1, n_sp, nchunk):
    rpt = n_sp // NS

    def body(dst_hbm, z_hbm, out_hbm, dst_v, ones_v, deg_sh):
        c = lax.axis_index("c")
        s = lax.axis_index("s")
        w = s * NC + c
        pltpu.sync_copy(z_hbm.at[pl.ds(s * rpt, rpt)], deg_sh.at[pl.ds(s * rpt, rpt)])
        for i in range(CH // 16):
            ones_v[pl.ds(i * 16, 16)] = jnp.full((16,), 1.0, jnp.float32)
        pltpu.sync_copy(dst_hbm.at[w], dst_v)
        plsc.subcore_barrier()

        def chunk(j, carry):
            pltpu.sync_copy(ones_v, deg_sh.at[dst_v.at[j]], add=True)
            return carry

        lax.fori_loop(0, nchunk, chunk, 0)
        plsc.subcore_barrier()
        pltpu.sync_copy(deg_sh.at[pl.ds(s * rpt, rpt)],
                        out_hbm.at[c].at[pl.ds(s * rpt, rpt)])

    f = pl.kernel(
        body,
        out_type=jax.ShapeDtypeStruct((NC, n_sp), jnp.float32),
        mesh=_sc_mesh(),
        scratch_types=[
            pltpu.VMEM((nchunk, CH), jnp.int32),
            pltpu.VMEM((CH,), jnp.float32),
            pltpu.VMEM_SHARED((n_sp,), jnp.float32),
        ],
    )
    return f(dst3, zeros1)


def _spmm_call(pk0, pk1, hs, zeros2, n_sp, d, n0, n1):
    rpt = n_sp // NS

    def body(pk0_hbm, pk1_hbm, hs_hbm, z_hbm, out_hbm, pk_v, sidx_v, didx_v,
             rows_v, agg_sh, gsem, ssem, psem0, psem1):
        c = lax.axis_index("c")
        s = lax.axis_index("s")
        sl = pl.ds(s * rpt, rpt)
        psem = (psem0, psem1)

        @pl.when(c == 0)
        def _():
            pltpu.sync_copy(z_hbm.at[sl], agg_sh.at[sl])

        @pl.when(c != 0)
        def _():
            pltpu.sync_copy(hs_hbm.at[sl], agg_sh.at[sl])

        plsc.subcore_barrier()

        def run(pk_hbm, nck):
            def pkload(j, m):
                pltpu.async_copy(pk_hbm.at[s].at[j], pk_v.at[m], psem[m])

            def pk_wait(j, m):
                pltpu.make_async_copy(pk_hbm.at[s].at[j], pk_v.at[m], psem[m]).wait()

            def unpack_idx(b, m):
                for i in range(CH // 16):
                    pkw = pk_v[m, pl.ds(i * 16, 16)]
                    sidx_v[b, pl.ds(i * 16, 16)] = lax.bitwise_and(pkw, 0xFFFF)
                    didx_v[b, pl.ds(i * 16, 16)] = lax.shift_right_logical(pkw, 16)

            def gather(b):
                pltpu.async_copy(hs_hbm.at[sidx_v.at[b]], rows_v.at[b], gsem)

            def gather_wait(b):
                pltpu.make_async_copy(hs_hbm.at[sidx_v.at[b]], rows_v.at[b], gsem).wait()

            def scatter(b):
                pltpu.async_copy(rows_v.at[b], agg_sh.at[didx_v.at[b]], ssem, add=True)

            def scatter_wait(b):
                pltpu.make_async_copy(rows_v.at[b], agg_sh.at[didx_v.at[b]], ssem).wait()

            pltpu.sync_copy(pk_hbm.at[s].at[0], pk_v.at[0])
            unpack_idx(0, 0)
            gather(0)
            pkload(1, 1)
            pkload(2, 0)

            def six(q, carry):
                for u in range(6):
                    j = q * 6 + u
                    b = u % 3
                    nb = (u + 1) % 3
                    m = (u + 1) % 2

                    @pl.when(j >= 2)
                    def _():
                        scatter_wait(nb)

                    @pl.when(j + 1 < nck)
                    def _():
                        pk_wait(j + 1, m)
                        unpack_idx(nb, m)
                        gather(nb)

                    @pl.when(j + 3 < nck)
                    def _():
                        pkload(j + 3, m)

                    gather_wait(b)
                    scatter(b)
                return carry

            lax.fori_loop(0, nck // 6, six, 0)
            scatter_wait((nck - 2) % 3)
            scatter_wait((nck - 1) % 3)

        @pl.when(c == 0)
        def _():
            run(pk0_hbm, n0)

        @pl.when(c != 0)
        def _():
            run(pk1_hbm, n1)

        plsc.subcore_barrier()
        pltpu.sync_copy(agg_sh.at[sl], out_hbm.at[c].at[sl])

    f = pl.kernel(
        body,
        out_type=jax.ShapeDtypeStruct((NC, n_sp, d), jnp.float32),
        mesh=_sc_mesh(),
        scratch_types=[
            pltpu.VMEM((2, CH), jnp.int32),
            pltpu.VMEM((3, CH), jnp.int32),
            pltpu.VMEM((3, CH), jnp.int32),
            pltpu.VMEM((3, CH, d), jnp.float32),
            pltpu.VMEM_SHARED((n_sp, d), jnp.float32),
            pltpu.SemaphoreType.DMA,
            pltpu.SemaphoreType.DMA,
            pltpu.SemaphoreType.DMA,
            pltpu.SemaphoreType.DMA,
        ],
    )
    return f(pk0, pk1, hs, zeros2)


def _tc_prologue(dega, degb, maskc, xp, W, n_sp, d, bn):
    def body(dega_ref, degb_ref, mask_ref, x_ref, w_ref, dinv_ref, hs_ref):
        deg = dega_ref[...] + degb_ref[...] + 1.0
        dinv = lax.rsqrt(deg) * mask_ref[...]
        dinv_ref[...] = dinv
        hs_ref[...] = jnp.dot(x_ref[...], w_ref[...],
                              preferred_element_type=jnp.float32) * dinv

    return pl.pallas_call(
        body,
        grid=(n_sp // bn,),
        in_specs=[
            pl.BlockSpec((bn, 1), lambda i: (i, 0)),
            pl.BlockSpec((bn, 1), lambda i: (i, 0)),
            pl.BlockSpec((bn, 1), lambda i: (i, 0)),
            pl.BlockSpec((bn, d), lambda i: (i, 0)),
            pl.BlockSpec((d, d), lambda i: (0, 0)),
        ],
        out_specs=[
            pl.BlockSpec((bn, 1), lambda i: (i, 0)),
            pl.BlockSpec((bn, d), lambda i: (i, 0)),
        ],
        out_shape=[
            jax.ShapeDtypeStruct((n_sp, 1), jnp.float32),
            jax.ShapeDtypeStruct((n_sp, d), jnp.float32),
        ],
    )(dega, degb, maskc, xp, W)


def _tc_mid(agg, dinv, b2d, W, n_sp, d, bn):
    def body(agga_ref, aggb_ref, dinv_ref, b_ref, w_ref, out_ref):
        g = (agga_ref[...] + aggb_ref[...]) * dinv_ref[...] + b_ref[...]
        h = jnp.maximum(g, 0.0)
        out_ref[...] = jnp.dot(h, w_ref[...],
                               preferred_element_type=jnp.float32) * dinv_ref[...]

    return pl.pallas_call(
        body,
        grid=(n_sp // bn,),
        in_specs=[
            pl.BlockSpec((bn, d), lambda i: (i, 0)),
            pl.BlockSpec((bn, d), lambda i: (i, 0)),
            pl.BlockSpec((bn, 1), lambda i: (i, 0)),
            pl.BlockSpec((1, d), lambda i: (0, 0)),
            pl.BlockSpec((d, d), lambda i: (0, 0)),
        ],
        out_specs=pl.BlockSpec((bn, d), lambda i: (i, 0)),
        out_shape=jax.ShapeDtypeStruct((n_sp, d), jnp.float32),
    )(agg[0], agg[1], dinv, b2d, W)


def _tc_epilogue(agg, dinv, maskc, b2d, Wl, bl2d, n_sp, d, n_real, bn):
    grid_n = n_sp // bn

    def body(agga_ref, aggb_ref, dinv_ref, mask_ref, b_ref, wl_ref, bl_ref, out_ref):
        i = pl.program_id(0)
        g = (agga_ref[...] + aggb_ref[...]) * dinv_ref[...] + b_ref[...]
        h = jnp.maximum(g, 0.0) * mask_ref[...]
        psum = jnp.sum(h, axis=0, keepdims=True)

        @pl.when(i == 0)
        def _():
            out_ref[...] = psum

        @pl.when(i > 0)
        def _():
            out_ref[...] += psum

        @pl.when(i == grid_n - 1)
        def _():
            pooled = out_ref[...] * (1.0 / n_real)
            out_ref[...] = lax.dot_general(
                pooled, wl_ref[...], (((1,), (1,)), ((), ())),
                preferred_element_type=jnp.float32) + bl_ref[...]

    return pl.pallas_call(
        body,
        grid=(grid_n,),
        in_specs=[
            pl.BlockSpec((bn, d), lambda i: (i, 0)),
            pl.BlockSpec((bn, d), lambda i: (i, 0)),
            pl.BlockSpec((bn, 1), lambda i: (i, 0)),
            pl.BlockSpec((bn, 1), lambda i: (i, 0)),
            pl.BlockSpec((1, d), lambda i: (0, 0)),
            pl.BlockSpec((d, d), lambda i: (0, 0)),
            pl.BlockSpec((1, d), lambda i: (0, 0)),
        ],
        out_specs=pl.BlockSpec((1, d), lambda i: (0, 0)),
        out_shape=jax.ShapeDtypeStruct((1, d), jnp.float32),
    )(agg[0], agg[1], dinv, maskc, b2d, Wl, bl2d)


def kernel(x, edge_index, W1, b1, W2, b2, W3, b3, Wl, bl):
    N, d = x.shape
    E = edge_index.shape[1]
    n_sp = ((N + 8 + NS * 128 - 1) // (NS * 128)) * (NS * 128)
    tchunk = -(-E // CH)
    n0 = max(6, int(round(tchunk * 0.835 / NS / 6)) * 6)
    n1 = max(6, -(-max(tchunk - n0 * NS, 1) // (NS * 6)) * 6)
    e_pad = (n0 + n1) * NS * CH
    bn = n_sp // NS
    while bn > 1600 or n_sp % bn or bn % 8:
        bn //= 2

    ei = edge_index.astype(jnp.int32)
    padv = jnp.full((e_pad - E,), N, jnp.int32)
    src_f = jnp.concatenate([ei[0], padv])
    dst_f = jnp.concatenate([ei[1], padv])
    packed2 = (src_f | (dst_f << 16)).reshape(-1, CH)
    pk0 = packed2[:NS * n0].reshape(NS, n0, CH)
    pk1 = packed2[NS * n0:].reshape(NS, n1, CH)
    nchunk = -(-tchunk // NW)
    nchunk = -(-nchunk // 2) * 2
    dst3 = jnp.concatenate(
        [dst_f[:E], jnp.full((NW * nchunk * CH - E,), N, jnp.int32)]
    ).reshape(NW, nchunk, CH)
    xp = jnp.zeros((n_sp, d), jnp.float32).at[:N, :].set(x)
    maskc = (jnp.arange(n_sp, dtype=jnp.int32) < N).astype(jnp.float32)[:, None]
    zeros2 = jnp.zeros((n_sp, d), jnp.float32)
    zeros1 = jnp.zeros((n_sp,), jnp.float32)

    deg2 = _deg_call(dst3, zeros1, n_sp, nchunk)
    dinv, hs1 = _tc_prologue(deg2[0][:, None], deg2[1][:, None], maskc, xp, W1,
                             n_sp, d, bn)
    agg1 = _spmm_call(pk0, pk1, hs1, zeros2, n_sp, d, n0, n1)
    hs2 = _tc_mid(agg1, dinv, b1[None, :], W2, n_sp, d, bn)
    agg2 = _spmm_call(pk0, pk1, hs2, zeros2, n_sp, d, n0, n1)
    hs3 = _tc_mid(agg2, dinv, b2[None, :], W3, n_sp, d, bn)
    agg3 = _spmm_call(pk0, pk1, hs3, zeros2, n_sp, d, n0, n1)
    return _tc_epilogue(agg3, dinv, maskc, b3[None, :], Wl,
                        bl[None, :], n_sp, d, N, bn)


# speedup vs baseline: 1.0667x; 1.0106x over previous
import functools

import jax
import jax.numpy as jnp
from jax import lax
from jax.experimental import pallas as pl
from jax.experimental.pallas import tpu as pltpu
from jax.experimental.pallas import tpu_sc as plsc

NC = 2
NS = 16
NW = NC * NS
CH = 112


def _sc_mesh():
    return plsc.VectorSubcoreMesh(
        core_axis_name="c", subcore_axis_name="s", num_cores=NC, num_subcores=NS
    )


def _deg_call(dst3, zeros1, n_sp, nchunk):
    rpt = n_sp // NS

    def body(dst_hbm, z_hbm, out_hbm, dst_v, ones_v, deg_sh):
        c = lax.axis_index("c")
        s = lax.axis_index("s")
        w = s * NC + c
        pltpu.sync_copy(z_hbm.at[pl.ds(s * rpt, rpt)], deg_sh.at[pl.ds(s * rpt, rpt)])
        for i in range(CH // 16):
            ones_v[pl.ds(i * 16, 16)] = jnp.full((16,), 1.0, jnp.float32)
        pltpu.sync_copy(dst_hbm.at[w], dst_v)
        plsc.subcore_barrier()

        def chunk(j, carry):
            pltpu.sync_copy(ones_v, deg_sh.at[dst_v.at[j]], add=True)
            return carry

        lax.fori_loop(0, nchunk, chunk, 0)
        plsc.subcore_barrier()
        pltpu.sync_copy(deg_sh.at[pl.ds(s * rpt, rpt)],
                        out_hbm.at[c].at[pl.ds(s * rpt, rpt)])

    f = pl.kernel(
        body,
        out_type=jax.ShapeDtypeStruct((NC, n_sp), jnp.float32),
        mesh=_sc_mesh(),
        scratch_types=[
            pltpu.VMEM((nchunk, CH), jnp.int32),
            pltpu.VMEM((CH,), jnp.float32),
            pltpu.VMEM_SHARED((n_sp,), jnp.float32),
        ],
    )
    return f(dst3, zeros1)


def _spmm_call(pk0, pk1, hs, zeros2, n_sp, d, n0, n1):
    rpt = n_sp // NS

    def body(pk0_hbm, pk1_hbm, hs_hbm, z_hbm, out_hbm, pk_v, sidx_v, didx_v,
             rows_v, agg_sh, gsem, ssem, psem0, psem1):
        c = lax.axis_index("c")
        s = lax.axis_index("s")
        sl = pl.ds(s * rpt, rpt)
        psem = (psem0, psem1)

        @pl.when(c == 0)
        def _():
            pltpu.sync_copy(z_hbm.at[sl], agg_sh.at[sl])

        @pl.when(c != 0)
        def _():
            pltpu.sync_copy(hs_hbm.at[sl], agg_sh.at[sl])

        plsc.subcore_barrier()

        def run(pk_hbm, nck):
            def pkload(j, m):
                pltpu.async_copy(pk_hbm.at[s].at[j], pk_v.at[m], psem[m])

            def pk_wait(j, m):
                pltpu.make_async_copy(pk_hbm.at[s].at[j], pk_v.at[m], psem[m]).wait()

            def unpack_idx(b, m):
                for i in range(CH // 16):
                    pkw = pk_v[m, pl.ds(i * 16, 16)]
                    sidx_v[b, pl.ds(i * 16, 16)] = lax.bitwise_and(pkw, 0xFFFF)
                    didx_v[b, pl.ds(i * 16, 16)] = lax.shift_right_logical(pkw, 16)

            def gather(b):
                pltpu.async_copy(hs_hbm.at[sidx_v.at[b]], rows_v.at[b], gsem)

            def gather_wait(b):
                pltpu.make_async_copy(hs_hbm.at[sidx_v.at[b]], rows_v.at[b], gsem).wait()

            def scatter(b):
                pltpu.async_copy(rows_v.at[b], agg_sh.at[didx_v.at[b]], ssem, add=True)

            def scatter_wait(b):
                pltpu.make_async_copy(rows_v.at[b], agg_sh.at[didx_v.at[b]], ssem).wait()

            pltpu.sync_copy(pk_hbm.at[s].at[0], pk_v.at[0])
            unpack_idx(0, 0)
            gather(0)
            pkload(1, 1)
            pkload(2, 0)

            def six(q, carry):
                for u in range(6):
                    j = q * 6 + u
                    b = u % 3
                    nb = (u + 1) % 3
                    m = (u + 1) % 2

                    @pl.when(j >= 2)
                    def _():
                        scatter_wait(nb)

                    @pl.when(j + 1 < nck)
                    def _():
                        pk_wait(j + 1, m)
                        unpack_idx(nb, m)
                        gather(nb)

                    @pl.when(j + 3 < nck)
                    def _():
                        pkload(j + 3, m)

                    gather_wait(b)
                    scatter(b)
                return carry

            lax.fori_loop(0, nck // 6, six, 0)
            scatter_wait((nck - 2) % 3)
            scatter_wait((nck - 1) % 3)

        @pl.when(c == 0)
        def _():
            run(pk0_hbm, n0)

        @pl.when(c != 0)
        def _():
            run(pk1_hbm, n1)

        plsc.subcore_barrier()
        pltpu.sync_copy(agg_sh.at[sl], out_hbm.at[c].at[sl])

    f = pl.kernel(
        body,
        out_type=jax.ShapeDtypeStruct((NC, n_sp, d), jnp.float32),
        mesh=_sc_mesh(),
        scratch_types=[
            pltpu.VMEM((2, CH), jnp.int32),
            pltpu.VMEM((3, CH), jnp.int32),
            pltpu.VMEM((3, CH), jnp.int32),
            pltpu.VMEM((3, CH, d), jnp.float32),
            pltpu.VMEM_SHARED((n_sp, d), jnp.float32),
            pltpu.SemaphoreType.DMA,
            pltpu.SemaphoreType.DMA,
            pltpu.SemaphoreType.DMA,
            pltpu.SemaphoreType.DMA,
        ],
    )
    return f(pk0, pk1, hs, zeros2)


def _tc_prologue(dega, degb, maskc, xp, W, n_sp, d, bn):
    def body(dega_ref, degb_ref, mask_ref, x_ref, w_ref, dinv_ref, hs_ref):
        deg = dega_ref[...] + degb_ref[...] + 1.0
        dinv = lax.rsqrt(deg) * mask_ref[...]
        dinv_ref[...] = dinv
        hs_ref[...] = jnp.dot(x_ref[...], w_ref[...],
                              preferred_element_type=jnp.float32) * dinv

    return pl.pallas_call(
        body,
        grid=(n_sp // bn,),
        in_specs=[
            pl.BlockSpec((bn, 1), lambda i: (i, 0)),
            pl.BlockSpec((bn, 1), lambda i: (i, 0)),
            pl.BlockSpec((bn, 1), lambda i: (i, 0)),
            pl.BlockSpec((bn, d), lambda i: (i, 0)),
            pl.BlockSpec((d, d), lambda i: (0, 0)),
        ],
        out_specs=[
            pl.BlockSpec((bn, 1), lambda i: (i, 0)),
            pl.BlockSpec((bn, d), lambda i: (i, 0)),
        ],
        out_shape=[
            jax.ShapeDtypeStruct((n_sp, 1), jnp.float32),
            jax.ShapeDtypeStruct((n_sp, d), jnp.float32),
        ],
    )(dega, degb, maskc, xp, W)


def _tc_mid(agg, dinv, b2d, W, n_sp, d, bn):
    def body(agga_ref, aggb_ref, dinv_ref, b_ref, w_ref, out_ref):
        g = (agga_ref[...] + aggb_ref[...]) * dinv_ref[...] + b_ref[...]
        h = jnp.maximum(g, 0.0)
        out_ref[...] = jnp.dot(h, w_ref[...],
                               preferred_element_type=jnp.float32) * dinv_ref[...]

    return pl.pallas_call(
        body,
        grid=(n_sp // bn,),
        in_specs=[
            pl.BlockSpec((bn, d), lambda i: (i, 0)),
            pl.BlockSpec((bn, d), lambda i: (i, 0)),
            pl.BlockSpec((bn, 1), lambda i: (i, 0)),
            pl.BlockSpec((1, d), lambda i: (0, 0)),
            pl.BlockSpec((d, d), lambda i: (0, 0)),
        ],
        out_specs=pl.BlockSpec((bn, d), lambda i: (i, 0)),
        out_shape=jax.ShapeDtypeStruct((n_sp, d), jnp.float32),
    )(agg[0], agg[1], dinv, b2d, W)


def _tc_epilogue(agg, dinv, maskc, b2d, Wl, bl2d, n_sp, d, n_real, bn):
    grid_n = n_sp // bn

    def body(agga_ref, aggb_ref, dinv_ref, mask_ref, b_ref, wl_ref, bl_ref, out_ref):
        i = pl.program_id(0)
        g = (agga_ref[...] + aggb_ref[...]) * dinv_ref[...] + b_ref[...]
        h = jnp.maximum(g, 0.0) * mask_ref[...]
        psum = jnp.sum(h, axis=0, keepdims=True)

        @pl.when(i == 0)
        def _():
            out_ref[...] = psum

        @pl.when(i > 0)
        def _():
            out_ref[...] += psum

        @pl.when(i == grid_n - 1)
        def _():
            pooled = out_ref[...] * (1.0 / n_real)
            out_ref[...] = lax.dot_general(
                pooled, wl_ref[...], (((1,), (1,)), ((), ())),
                preferred_element_type=jnp.float32) + bl_ref[...]

    return pl.pallas_call(
        body,
        grid=(grid_n,),
        in_specs=[
            pl.BlockSpec((bn, d), lambda i: (i, 0)),
            pl.BlockSpec((bn, d), lambda i: (i, 0)),
            pl.BlockSpec((bn, 1), lambda i: (i, 0)),
            pl.BlockSpec((bn, 1), lambda i: (i, 0)),
            pl.BlockSpec((1, d), lambda i: (0, 0)),
            pl.BlockSpec((d, d), lambda i: (0, 0)),
            pl.BlockSpec((1, d), lambda i: (0, 0)),
        ],
        out_specs=pl.BlockSpec((1, d), lambda i: (0, 0)),
        out_shape=jax.ShapeDtypeStruct((1, d), jnp.float32),
    )(agg[0], agg[1], dinv, maskc, b2d, Wl, bl2d)


def kernel(x, edge_index, W1, b1, W2, b2, W3, b3, Wl, bl):
    N, d = x.shape
    E = edge_index.shape[1]
    n_sp = ((N + 8 + NS * 128 - 1) // (NS * 128)) * (NS * 128)
    tchunk = -(-E // CH)
    n0 = max(6, int(round(tchunk * 0.873 / NS / 6)) * 6)
    n1 = max(6, -(-max(tchunk - n0 * NS, 1) // (NS * 6)) * 6)
    e_pad = (n0 + n1) * NS * CH
    bn = n_sp // NS
    while bn > 1600 or n_sp % bn or bn % 8:
        bn //= 2

    ei = edge_index.astype(jnp.int32)
    padv = jnp.full((e_pad - E,), N, jnp.int32)
    src_f = jnp.concatenate([ei[0], padv])
    dst_f = jnp.concatenate([ei[1], padv])
    packed2 = (src_f | (dst_f << 16)).reshape(-1, CH)
    pk0 = packed2[:NS * n0].reshape(NS, n0, CH)
    pk1 = packed2[NS * n0:].reshape(NS, n1, CH)
    nchunk = -(-tchunk // NW)
    nchunk = -(-nchunk // 2) * 2
    dst3 = jnp.concatenate(
        [dst_f[:E], jnp.full((NW * nchunk * CH - E,), N, jnp.int32)]
    ).reshape(NW, nchunk, CH)
    xp = jnp.zeros((n_sp, d), jnp.float32).at[:N, :].set(x)
    maskc = (jnp.arange(n_sp, dtype=jnp.int32) < N).astype(jnp.float32)[:, None]
    zeros2 = jnp.zeros((n_sp, d), jnp.float32)
    zeros1 = jnp.zeros((n_sp,), jnp.float32)

    deg2 = _deg_call(dst3, zeros1, n_sp, nchunk)
    dinv, hs1 = _tc_prologue(deg2[0][:, None], deg2[1][:, None], maskc, xp, W1,
                             n_sp, d, bn)
    agg1 = _spmm_call(pk0, pk1, hs1, zeros2, n_sp, d, n0, n1)
    hs2 = _tc_mid(agg1, dinv, b1[None, :], W2, n_sp, d, bn)
    agg2 = _spmm_call(pk0, pk1, hs2, zeros2, n_sp, d, n0, n1)
    hs3 = _tc_mid(agg2, dinv, b2[None, :], W3, n_sp, d, bn)
    agg3 = _spmm_call(pk0, pk1, hs3, zeros2, n_sp, d, n0, n1)
    return _tc_epilogue(agg3, dinv, maskc, b3[None, :], Wl,
                        bl[None, :], n_sp, d, N, bn)


# speedup vs baseline: 1.0675x; 1.0008x over previous
import jax
import jax.numpy as jnp
from jax import lax
from jax.experimental import pallas as pl
from jax.experimental.pallas import tpu as pltpu
from jax.experimental.pallas import tpu_sc as plsc

NC = 2
NS = 16
NW = NC * NS
CH = 112


def _sc_mesh():
    return plsc.VectorSubcoreMesh(
        core_axis_name="c", subcore_axis_name="s", num_cores=NC, num_subcores=NS
    )


def _deg_call(dst3, zeros1, n_sp, nchunk):
    rpt = n_sp // NS

    def body(dst_hbm, z_hbm, out_hbm, dst_v, ones_v, deg_sh):
        c = lax.axis_index("c")
        s = lax.axis_index("s")
        w = s * NC + c
        pltpu.sync_copy(z_hbm.at[pl.ds(s * rpt, rpt)], deg_sh.at[pl.ds(s * rpt, rpt)])
        for i in range(CH // 16):
            ones_v[pl.ds(i * 16, 16)] = jnp.full((16,), 1.0, jnp.float32)
        pltpu.sync_copy(dst_hbm.at[w], dst_v)
        plsc.subcore_barrier()

        def chunk(j, carry):
            pltpu.sync_copy(ones_v, deg_sh.at[dst_v.at[j]], add=True)
            return carry

        lax.fori_loop(0, nchunk, chunk, 0)
        plsc.subcore_barrier()
        pltpu.sync_copy(deg_sh.at[pl.ds(s * rpt, rpt)],
                        out_hbm.at[c].at[pl.ds(s * rpt, rpt)])

    f = pl.kernel(
        body,
        out_type=jax.ShapeDtypeStruct((NC, n_sp), jnp.float32),
        mesh=_sc_mesh(),
        scratch_types=[
            pltpu.VMEM((nchunk, CH), jnp.int32),
            pltpu.VMEM((CH,), jnp.float32),
            pltpu.VMEM_SHARED((n_sp,), jnp.float32),
        ],
    )
    return f(dst3, zeros1)


def _spmm_call(pk0, pk1, hs, zeros2, n_sp, d, n0, n1):
    rpt = n_sp // NS

    def body(pk0_hbm, pk1_hbm, hs_hbm, z_hbm, out_hbm, pk_v, sidx_v, didx_v,
             rows_v, agg_sh, gsem, ssem, psem0, psem1):
        c = lax.axis_index("c")
        s = lax.axis_index("s")
        sl = pl.ds(s * rpt, rpt)
        psem = (psem0, psem1)

        @pl.when(c == 0)
        def _():
            pltpu.sync_copy(z_hbm.at[sl], agg_sh.at[sl])

        @pl.when(c != 0)
        def _():
            pltpu.sync_copy(hs_hbm.at[sl], agg_sh.at[sl])

        plsc.subcore_barrier()

        def run(pk_hbm, nck):
            def pkload(j, m):
                pltpu.async_copy(pk_hbm.at[s].at[j], pk_v.at[m], psem[m])

            def pk_wait(j, m):
                pltpu.make_async_copy(pk_hbm.at[s].at[j], pk_v.at[m], psem[m]).wait()

            def unpack_idx(b, m):
                for i in range(CH // 16):
                    pkw = pk_v[m, pl.ds(i * 16, 16)]
                    sidx_v[b, pl.ds(i * 16, 16)] = lax.bitwise_and(pkw, 0xFFFF)
                    didx_v[b, pl.ds(i * 16, 16)] = lax.shift_right_logical(pkw, 16)

            def gather(b):
                pltpu.async_copy(hs_hbm.at[sidx_v.at[b]], rows_v.at[b], gsem)

            def gather_wait(b):
                pltpu.make_async_copy(hs_hbm.at[sidx_v.at[b]], rows_v.at[b], gsem).wait()

            def scatter(b):
                pltpu.async_copy(rows_v.at[b], agg_sh.at[didx_v.at[b]], ssem, add=True)

            def scatter_wait(b):
                pltpu.make_async_copy(rows_v.at[b], agg_sh.at[didx_v.at[b]], ssem).wait()

            pltpu.sync_copy(pk_hbm.at[s].at[0], pk_v.at[0])
            unpack_idx(0, 0)
            gather(0)
            pkload(1, 1)
            pkload(2, 0)

            def six(q, carry):
                for u in range(6):
                    j = q * 6 + u
                    b = u % 3
                    nb = (u + 1) % 3
                    m = (u + 1) % 2

                    @pl.when(j >= 2)
                    def _():
                        scatter_wait(nb)

                    @pl.when(j + 1 < nck)
                    def _():
                        pk_wait(j + 1, m)
                        unpack_idx(nb, m)
                        gather(nb)

                    @pl.when(j + 3 < nck)
                    def _():
                        pkload(j + 3, m)

                    gather_wait(b)
                    scatter(b)
                return carry

            lax.fori_loop(0, nck // 6, six, 0)
            scatter_wait((nck - 2) % 3)
            scatter_wait((nck - 1) % 3)

        @pl.when(c == 0)
        def _():
            run(pk0_hbm, n0)

        @pl.when(c != 0)
        def _():
            run(pk1_hbm, n1)

        plsc.subcore_barrier()
        pltpu.sync_copy(agg_sh.at[sl], out_hbm.at[c].at[sl])

    f = pl.kernel(
        body,
        out_type=jax.ShapeDtypeStruct((NC, n_sp, d), jnp.float32),
        mesh=_sc_mesh(),
        scratch_types=[
            pltpu.VMEM((2, CH), jnp.int32),
            pltpu.VMEM((3, CH), jnp.int32),
            pltpu.VMEM((3, CH), jnp.int32),
            pltpu.VMEM((3, CH, d), jnp.float32),
            pltpu.VMEM_SHARED((n_sp, d), jnp.float32),
            pltpu.SemaphoreType.DMA,
            pltpu.SemaphoreType.DMA,
            pltpu.SemaphoreType.DMA,
            pltpu.SemaphoreType.DMA,
        ],
    )
    return f(pk0, pk1, hs, zeros2)


def _tc_prologue(dega, degb, maskc, xp, W, n_sp, d, bn):
    def body(dega_ref, degb_ref, mask_ref, x_ref, w_ref, dinv_ref, hs_ref):
        deg = dega_ref[...] + degb_ref[...] + 1.0
        dinv = lax.rsqrt(deg) * mask_ref[...]
        dinv_ref[...] = dinv
        hs_ref[...] = jnp.dot(x_ref[...], w_ref[...],
                              preferred_element_type=jnp.float32) * dinv

    return pl.pallas_call(
        body,
        grid=(n_sp // bn,),
        in_specs=[
            pl.BlockSpec((bn, 1), lambda i: (i, 0)),
            pl.BlockSpec((bn, 1), lambda i: (i, 0)),
            pl.BlockSpec((bn, 1), lambda i: (i, 0)),
            pl.BlockSpec((bn, d), lambda i: (i, 0)),
            pl.BlockSpec((d, d), lambda i: (0, 0)),
        ],
        out_specs=[
            pl.BlockSpec((bn, 1), lambda i: (i, 0)),
            pl.BlockSpec((bn, d), lambda i: (i, 0)),
        ],
        out_shape=[
            jax.ShapeDtypeStruct((n_sp, 1), jnp.float32),
            jax.ShapeDtypeStruct((n_sp, d), jnp.float32),
        ],
    )(dega, degb, maskc, xp, W)


def _tc_mid(agg, dinv, b2d, W, n_sp, d, bn):
    def body(agga_ref, aggb_ref, dinv_ref, b_ref, w_ref, out_ref):
        g = (agga_ref[...] + aggb_ref[...]) * dinv_ref[...] + b_ref[...]
        h = jnp.maximum(g, 0.0)
        out_ref[...] = jnp.dot(h, w_ref[...],
                               preferred_element_type=jnp.float32) * dinv_ref[...]

    return pl.pallas_call(
        body,
        grid=(n_sp // bn,),
        in_specs=[
            pl.BlockSpec((bn, d), lambda i: (i, 0)),
            pl.BlockSpec((bn, d), lambda i: (i, 0)),
            pl.BlockSpec((bn, 1), lambda i: (i, 0)),
            pl.BlockSpec((1, d), lambda i: (0, 0)),
            pl.BlockSpec((d, d), lambda i: (0, 0)),
        ],
        out_specs=pl.BlockSpec((bn, d), lambda i: (i, 0)),
        out_shape=jax.ShapeDtypeStruct((n_sp, d), jnp.float32),
    )(agg[0], agg[1], dinv, b2d, W)


def _tc_epilogue(agg, dinv, maskc, b2d, Wl, bl2d, n_sp, d, n_real, bn):
    grid_n = n_sp // bn

    def body(agga_ref, aggb_ref, dinv_ref, mask_ref, b_ref, wl_ref, bl_ref, out_ref):
        i = pl.program_id(0)
        g = (agga_ref[...] + aggb_ref[...]) * dinv_ref[...] + b_ref[...]
        h = jnp.maximum(g, 0.0) * mask_ref[...]
        psum = jnp.sum(h, axis=0, keepdims=True)

        @pl.when(i == 0)
        def _():
            out_ref[...] = psum

        @pl.when(i > 0)
        def _():
            out_ref[...] += psum

        @pl.when(i == grid_n - 1)
        def _():
            pooled = out_ref[...] * (1.0 / n_real)
            out_ref[...] = lax.dot_general(
                pooled, wl_ref[...], (((1,), (1,)), ((), ())),
                preferred_element_type=jnp.float32) + bl_ref[...]

    return pl.pallas_call(
        body,
        grid=(grid_n,),
        in_specs=[
            pl.BlockSpec((bn, d), lambda i: (i, 0)),
            pl.BlockSpec((bn, d), lambda i: (i, 0)),
            pl.BlockSpec((bn, 1), lambda i: (i, 0)),
            pl.BlockSpec((bn, 1), lambda i: (i, 0)),
            pl.BlockSpec((1, d), lambda i: (0, 0)),
            pl.BlockSpec((d, d), lambda i: (0, 0)),
            pl.BlockSpec((1, d), lambda i: (0, 0)),
        ],
        out_specs=pl.BlockSpec((1, d), lambda i: (0, 0)),
        out_shape=jax.ShapeDtypeStruct((1, d), jnp.float32),
    )(agg[0], agg[1], dinv, maskc, b2d, Wl, bl2d)


def kernel(x, edge_index, W1, b1, W2, b2, W3, b3, Wl, bl):
    N, d = x.shape
    E = edge_index.shape[1]
    n_sp = ((N + 8 + NS * 128 - 1) // (NS * 128)) * (NS * 128)
    tchunk = -(-E // CH)
    n0 = max(6, int(round(tchunk * 0.873 / NS / 6)) * 6)
    n1 = max(6, -(-max(tchunk - n0 * NS, 1) // (NS * 6)) * 6)
    e_pad = (n0 + n1) * NS * CH
    bn = n_sp // NS
    while bn > 1600 or n_sp % bn or bn % 8:
        bn //= 2

    ei = edge_index.astype(jnp.int32)
    padv = jnp.full((e_pad - E,), N, jnp.int32)
    src_f = jnp.concatenate([ei[0], padv])
    dst_f = jnp.concatenate([ei[1], padv])
    packed2 = (src_f | (dst_f << 16)).reshape(-1, CH)
    pk0 = packed2[:NS * n0].reshape(NS, n0, CH)
    pk1 = packed2[NS * n0:].reshape(NS, n1, CH)
    nchunk = -(-tchunk // NW)
    nchunk = -(-nchunk // 2) * 2
    dst3 = jnp.concatenate(
        [dst_f[:E], jnp.full((NW * nchunk * CH - E,), N, jnp.int32)]
    ).reshape(NW, nchunk, CH)
    xp = jnp.zeros((n_sp, d), jnp.float32).at[:N, :].set(x)
    maskc = (jnp.arange(n_sp, dtype=jnp.int32) < N).astype(jnp.float32)[:, None]
    zeros2 = jnp.zeros((n_sp, d), jnp.float32)
    zeros1 = jnp.zeros((n_sp,), jnp.float32)

    deg2 = _deg_call(dst3, zeros1, n_sp, nchunk)
    dinv, hs1 = _tc_prologue(deg2[0][:, None], deg2[1][:, None], maskc, xp, W1,
                             n_sp, d, bn)
    agg1 = _spmm_call(pk0, pk1, hs1, zeros2, n_sp, d, n0, n1)
    hs2 = _tc_mid(agg1, dinv, b1[None, :], W2, n_sp, d, bn)
    agg2 = _spmm_call(pk0, pk1, hs2, zeros2, n_sp, d, n0, n1)
    hs3 = _tc_mid(agg2, dinv, b2[None, :], W3, n_sp, d, bn)
    agg3 = _spmm_call(pk0, pk1, hs3, zeros2, n_sp, d, n0, n1)
    return _tc_epilogue(agg3, dinv, maskc, b3[None, :], Wl,
                        bl[None, :], n_sp, d, N, bn)
